# Initial kernel scaffold; baseline (speedup 1.0000x reference)
import jax
import jax.numpy as jnp
from jax.experimental import pallas as pl


def kernel(x, w1, b1, w2, b2, w3, b3):
    raise NotImplementedError("write your pallas kernel here")


# speedup vs baseline: 2.1108x; 2.1108x over previous
import jax
import jax.numpy as jnp
from jax.experimental import pallas as pl
from jax.experimental.pallas import tpu as pltpu

_ACT = 2
_TB = 4096


def _mlp_kernel(x_ref, w1_ref, b1_ref, w2_ref, b2_ref, w3_ref, b3_ref, o_ref):
    x = x_ref[...]
    h1 = jnp.maximum(
        jnp.dot(x, w1_ref[...], preferred_element_type=jnp.float32) + b1_ref[...],
        0.0,
    )
    h2 = jnp.maximum(
        jnp.dot(h1, w2_ref[...], preferred_element_type=jnp.float32) + b2_ref[...],
        0.0,
    )
    o_ref[...] = (
        jnp.dot(h2, w3_ref[...], preferred_element_type=jnp.float32) + b3_ref[...]
    )


def kernel(x, w1, b1, w2, b2, w3, b3):
    B, F = x.shape
    w3s = w3[:, :_ACT]
    b3s = b3[:, :_ACT]

    b_pad8 = ((B + 7) // 8) * 8
    tb = min(_TB, b_pad8)
    b_pad = ((b_pad8 + tb - 1) // tb) * tb
    if b_pad != B:
        x = jnp.pad(x, ((0, b_pad - B), (0, 0)))

    const = lambda i: (0, 0)
    out = pl.pallas_call(
        _mlp_kernel,
        out_shape=jax.ShapeDtypeStruct((b_pad, _ACT), jnp.float32),
        grid=(b_pad // tb,),
        in_specs=[
            pl.BlockSpec((tb, F), lambda i: (i, 0)),
            pl.BlockSpec(w1.shape, const),
            pl.BlockSpec(b1.shape, const),
            pl.BlockSpec(w2.shape, const),
            pl.BlockSpec(b2.shape, const),
            pl.BlockSpec(w3s.shape, const),
            pl.BlockSpec(b3s.shape, const),
        ],
        out_specs=pl.BlockSpec((tb, _ACT), lambda i: (i, 0)),
        compiler_params=pltpu.CompilerParams(
            dimension_semantics=("parallel",),
        ),
    )(x, w1, b1, w2, b2, w3s, b3s)

    return out[:B]
